# Initial kernel scaffold; baseline (speedup 1.0000x reference)
import functools

import jax
import jax.numpy as jnp
from jax import lax
from jax.experimental import pallas as pl
from jax.experimental.pallas import tpu as pltpu
from jax.experimental.pallas import tpu_sc as plsc

N = 10000
E = 320000
G = 256
D = 128

NC = 2
NS = 16
NW = NC * NS

CH = 128
K = 79
EPW = K * CH
E_PAD = NW * EPW
N_PAD = 10240
RPT = N_PAD // NS
OPT = N // NS


def _sc_agg_body(x_hbm, src_hbm, dst_hbm, out0, out1,
                 src_v, dst_v, zbuf, gbuf, acc, sem):
    cid = lax.axis_index("c")
    sid = lax.axis_index("s")
    w = cid * NS + sid

    zeros16 = jnp.zeros((16,), jnp.float32)
    for r in range(16):
        for c in range(D // 16):
            zbuf[r, pl.ds(c * 16, 16)] = zeros16

    def zloop(k, carry):
        pltpu.sync_copy(zbuf, acc.at[pl.ds(sid * RPT + k * 16, 16)])
        return carry
    lax.fori_loop(0, RPT // 16, zloop, 0)

    pltpu.sync_copy(src_hbm.at[w], src_v)
    pltpu.sync_copy(dst_hbm.at[w], dst_v)

    plsc.subcore_barrier()

    def body(j, carry):
        pltpu.async_copy(x_hbm.at[src_v.at[j]], gbuf, sem).wait()
        pltpu.sync_copy(gbuf, acc.at[dst_v.at[j]], add=True)
        return carry
    lax.fori_loop(0, K, body, 0)

    plsc.subcore_barrier()

    def oloop(t, carry):
        r0 = sid * OPT + t * 125
        pltpu.sync_copy(acc.at[pl.ds(r0, 125)], gbuf.at[pl.ds(0, 125)])
        @pl.when(cid == 0)
        def _():
            pltpu.sync_copy(gbuf.at[pl.ds(0, 125)], out0.at[pl.ds(r0, 125)])
        @pl.when(cid == 1)
        def _():
            pltpu.sync_copy(gbuf.at[pl.ds(0, 125)], out1.at[pl.ds(r0, 125)])
        return carry
    lax.fori_loop(0, OPT // 125, oloop, 0)


_sc_agg = pl.kernel(
    _sc_agg_body,
    out_type=[jax.ShapeDtypeStruct((N, D), jnp.float32),
              jax.ShapeDtypeStruct((N, D), jnp.float32)],
    mesh=plsc.VectorSubcoreMesh(core_axis_name="c", subcore_axis_name="s",
                                num_cores=NC, num_subcores=NS),
    scratch_types=[
        pltpu.VMEM((K, CH), jnp.int32),
        pltpu.VMEM((K, CH), jnp.int32),
        pltpu.VMEM((16, D), jnp.float32),
        pltpu.VMEM((CH, D), jnp.float32),
        pltpu.VMEM_SHARED((N_PAD, D), jnp.float32),
        pltpu.SemaphoreType.DMA,
    ],
)


BLK = 2000


def _dense_relu_body(a0_ref, a1_ref, w_ref, b_ref, o_ref):
    x = a0_ref[...] + a1_ref[...]
    y = jnp.dot(x, w_ref[...], preferred_element_type=jnp.float32)
    o_ref[...] = jnp.maximum(y + b_ref[...], 0.0)


def _dense_relu(a0, a1, w, b):
    return pl.pallas_call(
        _dense_relu_body,
        grid=(N // BLK,),
        in_specs=[
            pl.BlockSpec((BLK, D), lambda i: (i, 0)),
            pl.BlockSpec((BLK, D), lambda i: (i, 0)),
            pl.BlockSpec((D, D), lambda i: (0, 0)),
            pl.BlockSpec((1, D), lambda i: (0, 0)),
        ],
        out_specs=pl.BlockSpec((BLK, D), lambda i: (i, 0)),
        out_shape=jax.ShapeDtypeStruct((N, D), jnp.float32),
    )(a0, a1, w, b.reshape(1, D))


def _final_body(a0_ref, a1_ref, h1_ref, h2_ref, pid_ref,
                w3_ref, b3_ref, wm1_ref, bm1_ref, wm2_ref, bm2_ref, o_ref):
    i = pl.program_id(0)
    h3 = (jnp.dot(a0_ref[...] + a1_ref[...], w3_ref[...],
                  preferred_element_type=jnp.float32) + b3_ref[...])
    u = (jnp.dot(h1_ref[...], wm1_ref[0:D], preferred_element_type=jnp.float32)
         + jnp.dot(h2_ref[...], wm1_ref[D:2 * D],
                   preferred_element_type=jnp.float32)
         + jnp.dot(h3, wm1_ref[2 * D:3 * D],
                   preferred_element_type=jnp.float32)
         + bm1_ref[...])
    u = jnp.maximum(u, 0.0)
    v = jnp.dot(u, wm2_ref[...], preferred_element_type=jnp.float32) + bm2_ref[...]
    onehot = (pid_ref[...] == lax.broadcasted_iota(jnp.int32, (1, G), 1)
              ).astype(jnp.float32)
    contrib = lax.dot_general(onehot, v, (((0,), (0,)), ((), ())),
                              preferred_element_type=jnp.float32)

    @pl.when(i == 0)
    def _():
        o_ref[...] = jnp.zeros_like(o_ref)
    o_ref[...] += contrib


def _final(a0, a1, h1, h2, pool2d, W3, b3, Wm1, bm1, Wm2, bm2):
    return pl.pallas_call(
        _final_body,
        grid=(N // BLK,),
        in_specs=[
            pl.BlockSpec((BLK, D), lambda i: (i, 0)),
            pl.BlockSpec((BLK, D), lambda i: (i, 0)),
            pl.BlockSpec((BLK, D), lambda i: (i, 0)),
            pl.BlockSpec((BLK, D), lambda i: (i, 0)),
            pl.BlockSpec((BLK, 1), lambda i: (i, 0)),
            pl.BlockSpec((D, D), lambda i: (0, 0)),
            pl.BlockSpec((1, D), lambda i: (0, 0)),
            pl.BlockSpec((3 * D, D), lambda i: (0, 0)),
            pl.BlockSpec((1, D), lambda i: (0, 0)),
            pl.BlockSpec((D, D), lambda i: (0, 0)),
            pl.BlockSpec((1, D), lambda i: (0, 0)),
        ],
        out_specs=pl.BlockSpec((G, D), lambda i: (0, 0)),
        out_shape=jax.ShapeDtypeStruct((G, D), jnp.float32),
    )(a0, a1, h1, h2, pool2d, W3, b3.reshape(1, D), Wm1,
      bm1.reshape(1, D), Wm2, bm2.reshape(1, D))


def kernel(features, edge_index, pool_ids,
           W1, b1, W2, b2, W3, b3, Wm1, bm1, Wm2, bm2):
    src = edge_index[0].astype(jnp.int32)
    dst = edge_index[1].astype(jnp.int32)
    pad = E_PAD - E
    src_w = jnp.concatenate(
        [src, jnp.zeros((pad,), jnp.int32)]).reshape(NW, K, CH)
    dst_w = jnp.concatenate(
        [dst, jnp.full((pad,), N, jnp.int32)]).reshape(NW, K, CH)
    pool2d = pool_ids.astype(jnp.int32).reshape(N, 1)

    a1a, a1b = _sc_agg(features, src_w, dst_w)
    h1 = _dense_relu(a1a, a1b, W1, b1)
    a2a, a2b = _sc_agg(h1, src_w, dst_w)
    h2 = _dense_relu(a2a, a2b, W2, b2)
    a3a, a3b = _sc_agg(h2, src_w, dst_w)
    return _final(a3a, a3b, h1, h2, pool2d, W3, b3, Wm1, bm1, Wm2, bm2)


# speedup vs baseline: 4.7562x; 4.7562x over previous
import functools

import jax
import jax.numpy as jnp
from jax import lax
from jax.experimental import pallas as pl
from jax.experimental.pallas import tpu as pltpu
from jax.experimental.pallas import tpu_sc as plsc

N = 10000
E = 320000
G = 256
D = 128

NC = 2
NS = 16
NW = NC * NS

CH = 128
K = 79
EPW = K * CH
E_PAD = NW * EPW
N_PAD = 10240
RPT = N_PAD // NS
OPT = N // NS


def _sc_agg_body(x_hbm, src_hbm, dst_hbm, out0, out1,
                 src_v, dst_v, zbuf, gbuf, acc, sem):
    cid = lax.axis_index("c")
    sid = lax.axis_index("s")
    w = cid * NS + sid

    zeros16 = jnp.zeros((16,), jnp.float32)
    for r in range(16):
        for c in range(D // 16):
            zbuf[r, pl.ds(c * 16, 16)] = zeros16

    def zloop(k, carry):
        pltpu.sync_copy(zbuf, acc.at[pl.ds(sid * RPT + k * 16, 16)])
        return carry
    lax.fori_loop(0, RPT // 16, zloop, 0)

    pltpu.sync_copy(src_hbm.at[w], src_v)
    pltpu.sync_copy(dst_hbm.at[w], dst_v)

    plsc.subcore_barrier()

    def body(j, carry):
        pltpu.async_copy(x_hbm.at[src_v.at[j]], gbuf, sem).wait()
        pltpu.sync_copy(gbuf, acc.at[dst_v.at[j]], add=True)
        return carry
    lax.fori_loop(0, K, body, 0)

    plsc.subcore_barrier()

    def oloop(t, carry):
        r0 = sid * RPT + t * CH
        pltpu.sync_copy(acc.at[pl.ds(r0, CH)], gbuf)
        @pl.when(cid == 0)
        def _():
            pltpu.sync_copy(gbuf, out0.at[pl.ds(r0, CH)])
        @pl.when(cid == 1)
        def _():
            pltpu.sync_copy(gbuf, out1.at[pl.ds(r0, CH)])
        return carry
    lax.fori_loop(0, RPT // CH, oloop, 0)


_sc_agg = pl.kernel(
    _sc_agg_body,
    out_type=[jax.ShapeDtypeStruct((N_PAD, D), jnp.float32),
              jax.ShapeDtypeStruct((N_PAD, D), jnp.float32)],
    mesh=plsc.VectorSubcoreMesh(core_axis_name="c", subcore_axis_name="s",
                                num_cores=NC, num_subcores=NS),
    scratch_types=[
        pltpu.VMEM((K, CH), jnp.int32),
        pltpu.VMEM((K, CH), jnp.int32),
        pltpu.VMEM((16, D), jnp.float32),
        pltpu.VMEM((CH, D), jnp.float32),
        pltpu.VMEM_SHARED((N_PAD, D), jnp.float32),
        pltpu.SemaphoreType.DMA,
    ],
)


BLK = 2000


def _dense_relu_body(a0_ref, a1_ref, w_ref, b_ref, o_ref):
    x = a0_ref[...] + a1_ref[...]
    y = jnp.dot(x, w_ref[...], preferred_element_type=jnp.float32)
    o_ref[...] = jnp.maximum(y + b_ref[...], 0.0)


def _dense_relu(a0, a1, w, b):
    return pl.pallas_call(
        _dense_relu_body,
        grid=(N // BLK,),
        in_specs=[
            pl.BlockSpec((BLK, D), lambda i: (i, 0)),
            pl.BlockSpec((BLK, D), lambda i: (i, 0)),
            pl.BlockSpec((D, D), lambda i: (0, 0)),
            pl.BlockSpec((1, D), lambda i: (0, 0)),
        ],
        out_specs=pl.BlockSpec((BLK, D), lambda i: (i, 0)),
        out_shape=jax.ShapeDtypeStruct((N, D), jnp.float32),
    )(a0, a1, w, b.reshape(1, D))


def _final_body(a0_ref, a1_ref, h1_ref, h2_ref, pid_ref,
                w3_ref, b3_ref, wm1_ref, bm1_ref, wm2_ref, bm2_ref, o_ref):
    i = pl.program_id(0)
    h3 = (jnp.dot(a0_ref[...] + a1_ref[...], w3_ref[...],
                  preferred_element_type=jnp.float32) + b3_ref[...])
    u = (jnp.dot(h1_ref[...], wm1_ref[0:D], preferred_element_type=jnp.float32)
         + jnp.dot(h2_ref[...], wm1_ref[D:2 * D],
                   preferred_element_type=jnp.float32)
         + jnp.dot(h3, wm1_ref[2 * D:3 * D],
                   preferred_element_type=jnp.float32)
         + bm1_ref[...])
    u = jnp.maximum(u, 0.0)
    v = jnp.dot(u, wm2_ref[...], preferred_element_type=jnp.float32) + bm2_ref[...]
    onehot = (pid_ref[...] == lax.broadcasted_iota(jnp.int32, (1, G), 1)
              ).astype(jnp.float32)
    contrib = lax.dot_general(onehot, v, (((0,), (0,)), ((), ())),
                              preferred_element_type=jnp.float32)

    @pl.when(i == 0)
    def _():
        o_ref[...] = jnp.zeros_like(o_ref)
    o_ref[...] += contrib


def _final(a0, a1, h1, h2, pool2d, W3, b3, Wm1, bm1, Wm2, bm2):
    return pl.pallas_call(
        _final_body,
        grid=(N // BLK,),
        in_specs=[
            pl.BlockSpec((BLK, D), lambda i: (i, 0)),
            pl.BlockSpec((BLK, D), lambda i: (i, 0)),
            pl.BlockSpec((BLK, D), lambda i: (i, 0)),
            pl.BlockSpec((BLK, D), lambda i: (i, 0)),
            pl.BlockSpec((BLK, 1), lambda i: (i, 0)),
            pl.BlockSpec((D, D), lambda i: (0, 0)),
            pl.BlockSpec((1, D), lambda i: (0, 0)),
            pl.BlockSpec((3 * D, D), lambda i: (0, 0)),
            pl.BlockSpec((1, D), lambda i: (0, 0)),
            pl.BlockSpec((D, D), lambda i: (0, 0)),
            pl.BlockSpec((1, D), lambda i: (0, 0)),
        ],
        out_specs=pl.BlockSpec((G, D), lambda i: (0, 0)),
        out_shape=jax.ShapeDtypeStruct((G, D), jnp.float32),
    )(a0, a1, h1, h2, pool2d, W3, b3.reshape(1, D), Wm1,
      bm1.reshape(1, D), Wm2, bm2.reshape(1, D))


def kernel(features, edge_index, pool_ids,
           W1, b1, W2, b2, W3, b3, Wm1, bm1, Wm2, bm2):
    src = edge_index[0].astype(jnp.int32)
    dst = edge_index[1].astype(jnp.int32)
    pad = E_PAD - E
    src_w = jnp.concatenate(
        [src, jnp.zeros((pad,), jnp.int32)]).reshape(NW, K, CH)
    dst_w = jnp.concatenate(
        [dst, jnp.full((pad,), N, jnp.int32)]).reshape(NW, K, CH)
    pool2d = pool_ids.astype(jnp.int32).reshape(N, 1)

    a1a, a1b = _sc_agg(features, src_w, dst_w)
    h1 = _dense_relu(a1a, a1b, W1, b1)
    a2a, a2b = _sc_agg(h1, src_w, dst_w)
    h2 = _dense_relu(a2a, a2b, W2, b2)
    a3a, a3b = _sc_agg(h2, src_w, dst_w)
    return _final(a3a, a3b, h1, h2, pool2d, W3, b3, Wm1, bm1, Wm2, bm2)
